# baseline (speedup 1.0000x reference)
import functools

import jax
import jax.numpy as jnp
from jax import lax
from jax.experimental import pallas as pl
from jax.experimental.pallas import tpu as pltpu
from jax.experimental.pallas import tpu_sc as plsc

NC = 2
NS = 16
NW = NC * NS

L = 16
BLK = 128
NBUF = 2
PADD = 128


@functools.cache
def _build(V, D, B0, B1):
  assert B0 % (NW * BLK) == 0
  rows_per_w = B0 // NW
  blk_per_w = rows_per_w // BLK
  n_blocks = B1 * blk_per_w
  DH = D // 8
  mesh = plsc.VectorSubcoreMesh(
      core_axis_name="c", subcore_axis_name="s", num_cores=NC, num_subcores=NS)

  @functools.partial(
      pl.kernel,
      out_type=jax.ShapeDtypeStruct((B1, DH, B0 // BLK, 8, BLK), jnp.float32),
      mesh=mesh,
      scratch_types=[
          pltpu.VMEM((rows_per_w, B1), jnp.int32),
          pltpu.VMEM((B1, rows_per_w), jnp.int32),
          [pltpu.VMEM((BLK, D), jnp.float32) for _ in range(NBUF)],
          [pltpu.VMEM((DH, 1, 8, BLK), jnp.float32) for _ in range(NBUF)],
          [pltpu.SemaphoreType.DMA for _ in range(NBUF)],
          [pltpu.SemaphoreType.DMA for _ in range(NBUF)],
      ],
      compiler_params=pltpu.CompilerParams(
          use_tc_tiling_on_sc=False, needs_layout_passes=False),
  )
  def gather_kernel(table_hbm, data_hbm, out_hbm, idx_v, idx_t, rows, tbuf,
                    gsem, ssem):
    wid = lax.axis_index("s") * NC + lax.axis_index("c")
    row0 = wid * rows_per_w
    pltpu.sync_copy(data_hbm.at[pl.ds(row0, rows_per_w)], idx_v)

    iota = lax.iota(jnp.int32, L)

    def idxt_body(j, carry):
      cols = jnp.full((L,), j, jnp.int32)
      for c in range(rows_per_w // L):
        vals = plsc.load_gather(idx_v, [iota + c * L, cols])
        idx_t[j, pl.ds(c * L, L)] = vals
      return carry

    lax.fori_loop(0, B1, idxt_body, 0)

    def jafor(t):
      j = t // blk_per_w
      a = t - j * blk_per_w
      return j, a

    def fire_block(t, b):
      j, a = jafor(t)
      pltpu.async_copy(
          table_hbm.at[idx_t.at[j, pl.ds(a * BLK, BLK)]], rows[b], gsem[b])

    def drain_block(t, b):
      j, a = jafor(t)
      pltpu.make_async_copy(
          table_hbm.at[idx_t.at[j, pl.ds(a * BLK, BLK)]], rows[b],
          gsem[b]).wait()

    def out_slice(t):
      j, a = jafor(t)
      ih = wid * blk_per_w + a
      return out_hbm.at[j, pl.ds(0, DH), pl.ds(ih, 1), pl.ds(0, 8),
                        pl.ds(0, BLK)]

    def store_block(t, b):
      pltpu.async_copy(tbuf[b], out_slice(t), ssem[b])

    def wait_store(t, b):
      pltpu.make_async_copy(tbuf[b], out_slice(t), ssem[b]).wait()

    def transpose_block(b):
      def dh_body(dh, carry):
        for dl in range(8):
          d = jnp.full((L,), dh * 8 + dl, jnp.int32)
          for c in range(BLK // L):
            vals = plsc.load_gather(rows[b], [iota + c * L, d])
            tbuf[b][dh, 0, dl, pl.ds(c * L, L)] = vals
        return carry

      lax.fori_loop(0, DH, dh_body, 0)

    for b in range(NBUF):
      fire_block(b, b)

    def step(p, carry):
      for b in range(NBUF):
        t = p * NBUF + b
        drain_block(t, b)

        @pl.when(t >= NBUF)
        def _():
          wait_store(t - NBUF, b)

        transpose_block(b)

        @pl.when(t + NBUF < n_blocks)
        def _():
          fire_block(t + NBUF, b)

        store_block(t, b)

      return carry

    lax.fori_loop(0, n_blocks // NBUF, step, 0)
    for b in range(NBUF):
      wait_store(n_blocks - NBUF + b, b)

  return gather_kernel


def kernel(data, ivectors):
  B0, B1 = data.shape
  V, D = ivectors.shape
  tpad = jnp.pad(ivectors, ((0, 0), (0, PADD - D))).reshape(V * (PADD // D), D)
  data2 = data.astype(jnp.int32) * (PADD // D)
  out = _build(V, D, B0, B1)(tpad, data2)
  return out.transpose(2, 4, 0, 1, 3).reshape(B0, B1, D)


# speedup vs baseline: 2.0502x; 2.0502x over previous
import functools

import jax
import jax.numpy as jnp
from jax import lax
from jax.experimental import pallas as pl
from jax.experimental.pallas import tpu as pltpu
from jax.experimental.pallas import tpu_sc as plsc

NC = 2
NS = 16
NW = NC * NS

G = 8
NBUF = 4
PADR = 56
PADD = 128


@functools.cache
def _build(V, D, B0, B1):
  assert B0 % NW == 0
  rows_per_w = B0 // NW
  assert rows_per_w % (G * NBUF) == 0
  n_groups = rows_per_w // G
  mesh = plsc.VectorSubcoreMesh(
      core_axis_name="c", subcore_axis_name="s", num_cores=NC, num_subcores=NS)

  @functools.partial(
      pl.kernel,
      out_type=jax.ShapeDtypeStruct((B0, PADR, PADD), jnp.float32),
      mesh=mesh,
      scratch_types=[
          pltpu.VMEM((rows_per_w, B1), jnp.int32),
          [pltpu.VMEM((G, B1, D), jnp.float32) for _ in range(NBUF)],
          [pltpu.SemaphoreType.DMA for _ in range(NBUF)],
          [pltpu.SemaphoreType.DMA for _ in range(NBUF)],
      ],
      compiler_params=pltpu.CompilerParams(use_tc_tiling_on_sc=False),
  )
  def gather_kernel(table_hbm, data_hbm, out_hbm, idx_v, rows, gsem, ssem):
    wid = lax.axis_index("s") * NC + lax.axis_index("c")
    row0 = wid * rows_per_w
    pltpu.sync_copy(data_hbm.at[pl.ds(row0, rows_per_w)], idx_v)

    def out_slice(g):
      return out_hbm.at[pl.ds(row0 + g * G, G), pl.ds(0, B1), pl.ds(0, D)]

    def fire_group(g, b):
      for k in range(G):
        pltpu.async_copy(
            table_hbm.at[idx_v.at[g * G + k]], rows[b].at[k], gsem[b])

    def drain_group(g, b):
      for k in range(G):
        pltpu.make_async_copy(
            table_hbm.at[idx_v.at[g * G + k]], rows[b].at[k], gsem[b]).wait()

    def store_group(g, b):
      pltpu.async_copy(rows[b], out_slice(g), ssem[b])

    def wait_store(g, b):
      pltpu.make_async_copy(rows[b], out_slice(g), ssem[b]).wait()

    for b in range(NBUF):
      fire_group(b, b)

    def step(p, carry):
      for b in range(NBUF):
        g = p * NBUF + b
        drain_group(g, b)
        store_group(g, b)

        @pl.when(g + NBUF < n_groups)
        def _():
          wait_store(g, b)
          fire_group(g + NBUF, b)

      return carry

    lax.fori_loop(0, n_groups // NBUF, step, 0)
    for b in range(NBUF):
      wait_store(n_groups - NBUF + b, b)

  return gather_kernel


def kernel(data, ivectors):
  B0, B1 = data.shape
  V, D = ivectors.shape
  tpad = jnp.pad(ivectors, ((0, 0), (0, PADD - D))).reshape(V * (PADD // D), D)
  data2 = data.astype(jnp.int32) * (PADD // D)
  out = _build(V, D, B0, B1)(tpad, data2)
  return out[:, :B1, :D]
